# baseline (speedup 1.0000x reference)
import jax
import jax.numpy as jnp
from jax import lax
from jax.experimental import pallas as pl
from jax.experimental.pallas import tpu as pltpu
from jax.experimental.pallas import tpu_sc as plsc

NC = 2
NS = 16
NW = NC * NS
LANE = 16
CHUNK = 128
TCH = 32
DEGW = 16
NB = 4
KD = 8


def _mesh():
    return plsc.VectorSubcoreMesh(
        core_axis_name="c", subcore_axis_name="s", num_cores=NC, num_subcores=NS
    )


def _deg_body(dst3, out, idx_v, ones_v, zrow_v, acc, dsem):
    c = lax.axis_index("c")
    s = lax.axis_index("s")
    w = c * NS + s
    chw = idx_v.shape[0]
    rpt = zrow_v.shape[0]

    pltpu.sync_copy(dst3.at[w], idx_v)

    @pl.loop(0, CHUNK)
    def _(r):
        ones_v[r, :] = jnp.ones((DEGW,), jnp.float32)

    @pl.loop(0, rpt)
    def _(r):
        zrow_v[r, :] = jnp.zeros((DEGW,), jnp.float32)

    pltpu.sync_copy(zrow_v, acc.at[pl.ds(s * rpt, rpt)])
    plsc.subcore_barrier()

    for j in range(KD):
        pltpu.async_copy(ones_v, acc.at[idx_v.at[j]], dsem, add=True)

    @pl.loop(0, chw - KD)
    def _(j):
        pltpu.async_copy(ones_v, acc.at[idx_v.at[j + KD]], dsem, add=True)
        pltpu.make_async_copy(ones_v, acc.at[idx_v.at[0]], dsem).wait()

    for _j in range(KD):
        pltpu.make_async_copy(ones_v, acc.at[idx_v.at[0]], dsem).wait()

    plsc.subcore_barrier()
    pltpu.sync_copy(acc.at[pl.ds(s * rpt, rpt)], out.at[c, pl.ds(s * rpt, rpt)])


def _agg_body(y, src3, dst3, out, idxs_v, idxd_v, rows_v, zbuf_v, acc,
              g0, g1, g2, g3, s0, s1, s2, s3):
    gsems = (g0, g1, g2, g3)
    ssems = (s0, s1, s2, s3)
    c = lax.axis_index("c")
    s = lax.axis_index("s")
    w = c * NS + s
    chw = idxs_v.shape[0]
    n_acc, d = acc.shape
    rpt = n_acc // NS
    zr = zbuf_v.shape[0]

    pltpu.sync_copy(src3.at[w], idxs_v)
    pltpu.sync_copy(dst3.at[w], idxd_v)

    @pl.loop(0, zr)
    def _(r):
        for q in range(d // LANE):
            zbuf_v[r, pl.ds(q * LANE, LANE)] = jnp.zeros((LANE,), jnp.float32)

    for k in range(rpt // zr):
        pltpu.async_copy(zbuf_v, acc.at[pl.ds(s * rpt + k * zr, zr)], g0)
    for k in range(rpt // zr):
        pltpu.make_async_copy(zbuf_v, acc.at[pl.ds(s * rpt, zr)], g0).wait()
    plsc.subcore_barrier()

    def sidx(idx, g, q):
        return idx.at[g, pl.ds(TCH * q, TCH)]

    def start_gather(g, q):
        pltpu.async_copy(y.at[sidx(idxs_v, g, q)], rows_v.at[q], gsems[q])

    def wait_gather(q):
        pltpu.make_async_copy(
            y.at[sidx(idxs_v, 0, 0)], rows_v.at[q], gsems[q]).wait()

    def start_scatter(g, q):
        pltpu.async_copy(
            rows_v.at[q], acc.at[sidx(idxd_v, g, q)], ssems[q], add=True)

    def wait_scatter(q):
        pltpu.make_async_copy(
            rows_v.at[q], acc.at[sidx(idxd_v, 0, 0)], ssems[q]).wait()

    for q in range(NB):
        start_gather(0, q)
    for q in range(NB):
        wait_gather(q)
        start_scatter(0, q)

    @pl.loop(1, chw)
    def _(g):
        for q in range(NB):
            wait_scatter(q)
            start_gather(g, q)
            wait_gather(q)
            start_scatter(g, q)

    for q in range(NB):
        wait_scatter(q)

    plsc.subcore_barrier()
    for k in range(rpt // CHUNK):
        off = s * rpt + k * CHUNK
        pltpu.async_copy(acc.at[pl.ds(off, CHUNK)], out.at[c, pl.ds(off, CHUNK)], g0)
    for k in range(rpt // CHUNK):
        pltpu.make_async_copy(
            acc.at[pl.ds(s * rpt, CHUNK)], out.at[c, pl.ds(s * rpt, CHUNK)], g0).wait()


_SC_LINEAR = pltpu.CompilerParams(use_tc_tiling_on_sc=False)


def _count_degrees(dst3, n_acc):
    chw = dst3.shape[1]
    rpt = n_acc // NS
    return pl.kernel(
        _deg_body,
        out_type=jax.ShapeDtypeStruct((NC, n_acc, DEGW), jnp.float32),
        mesh=_mesh(),
        compiler_params=_SC_LINEAR,
        scratch_types=[
            pltpu.VMEM((chw, CHUNK), jnp.int32),
            pltpu.VMEM((CHUNK, DEGW), jnp.float32),
            pltpu.VMEM((rpt, DEGW), jnp.float32),
            pltpu.VMEM_SHARED((n_acc, DEGW), jnp.float32),
            pltpu.SemaphoreType.DMA,
        ],
    )(dst3)


def _aggregate(y, src3, dst3, n_acc):
    chw = src3.shape[1]
    d = y.shape[1]
    return pl.kernel(
        _agg_body,
        out_type=jax.ShapeDtypeStruct((NC, n_acc, d), jnp.float32),
        mesh=_mesh(),
        scratch_types=[
            pltpu.VMEM((chw, CHUNK), jnp.int32),
            pltpu.VMEM((chw, CHUNK), jnp.int32),
            pltpu.VMEM((NB, TCH, d), jnp.float32),
            pltpu.VMEM((64, d), jnp.float32),
            pltpu.VMEM_SHARED((n_acc, d), jnp.float32),
        ] + [pltpu.SemaphoreType.DMA] * (2 * NB),
    )(y, src3, dst3)


def _row_block(n):
    for bm in (2000, 1024, 1000, 800, 640, 512, 400, 256, 200, 128, 80, 40, 8):
        if n % bm == 0:
            return bm
    return n


def _dinv(deg_ref, bm):
    deg = deg_ref[0, :, 0:1] + deg_ref[1, :, 0:1] + 1.0
    return lax.rsqrt(deg)


def _first_layer(x, w1, degp):
    n, d_in = x.shape
    d_h = w1.shape[1]
    bm = _row_block(n)

    def body(deg_ref, x_ref, w_ref, y_ref):
        y_ref[...] = jnp.dot(
            x_ref[...], w_ref[...],
            preferred_element_type=jnp.float32,
            precision=lax.Precision.HIGHEST,
        ) * _dinv(deg_ref, bm)

    return pl.pallas_call(
        body,
        grid=(n // bm,),
        in_specs=[
            pl.BlockSpec((NC, bm, DEGW), lambda i: (0, i, 0)),
            pl.BlockSpec((bm, d_in), lambda i: (i, 0)),
            pl.BlockSpec((d_in, d_h), lambda i: (0, 0)),
        ],
        out_specs=pl.BlockSpec((bm, d_h), lambda i: (i, 0)),
        out_shape=jax.ShapeDtypeStruct((n, d_h), jnp.float32),
    )(degp, x, w1)


def _mid_layer(aggp, y1, degp, b1, w2):
    n, d = y1.shape
    bm = _row_block(n)

    def body(agg_ref, y_ref, deg_ref, b_ref, w_ref, y2_ref):
        dinv = _dinv(deg_ref, bm)
        h = dinv * (agg_ref[0] + agg_ref[1] + y_ref[...]) + b_ref[...]
        h = jnp.maximum(h, 0.0)
        y2_ref[...] = jnp.dot(
            h, w_ref[...],
            preferred_element_type=jnp.float32,
            precision=lax.Precision.HIGHEST,
        ) * dinv

    return pl.pallas_call(
        body,
        grid=(n // bm,),
        in_specs=[
            pl.BlockSpec((NC, bm, d), lambda i: (0, i, 0)),
            pl.BlockSpec((bm, d), lambda i: (i, 0)),
            pl.BlockSpec((NC, bm, DEGW), lambda i: (0, i, 0)),
            pl.BlockSpec((1, d), lambda i: (0, 0)),
            pl.BlockSpec((d, d), lambda i: (0, 0)),
        ],
        out_specs=pl.BlockSpec((bm, d), lambda i: (i, 0)),
        out_shape=jax.ShapeDtypeStruct((n, d), jnp.float32),
    )(aggp, y1, degp, b1, w2)


def _final_layer(aggp, y2, degp, b2):
    n, d = y2.shape
    bm = _row_block(n)

    def body(agg_ref, y_ref, deg_ref, b_ref, h_ref):
        h = _dinv(deg_ref, bm) * (agg_ref[0] + agg_ref[1] + y_ref[...]) \
            + b_ref[...]
        h_ref[...] = jnp.maximum(h, 0.0)

    return pl.pallas_call(
        body,
        grid=(n // bm,),
        in_specs=[
            pl.BlockSpec((NC, bm, d), lambda i: (0, i, 0)),
            pl.BlockSpec((bm, d), lambda i: (i, 0)),
            pl.BlockSpec((NC, bm, DEGW), lambda i: (0, i, 0)),
            pl.BlockSpec((1, d), lambda i: (0, 0)),
        ],
        out_specs=pl.BlockSpec((bm, d), lambda i: (i, 0)),
        out_shape=jax.ShapeDtypeStruct((n, d), jnp.float32),
    )(aggp, y2, degp, b2)


def kernel(x, edge_index, W1, b1, W2, b2):
    n = x.shape[0]
    e = edge_index.shape[1]

    grp = NW * CHUNK
    e_pad = ((e + grp - 1) // grp) * grp
    chw = e_pad // (NW * CHUNK)
    n_acc = ((n + 1 + NS * CHUNK - 1) // (NS * CHUNK)) * (NS * CHUNK)

    idt = edge_index.dtype
    pad_i = jnp.arange(e_pad - e, dtype=idt)
    src3 = jnp.concatenate(
        [edge_index[0], pad_i % n]).reshape(NW, chw, CHUNK)
    dst3 = jnp.concatenate(
        [edge_index[1], n + pad_i % (n_acc - n)]).reshape(NW, chw, CHUNK)

    degp = _count_degrees(dst3, n_acc)
    y1 = _first_layer(x, W1, degp)
    a1 = _aggregate(y1, src3, dst3, n_acc)
    y2 = _mid_layer(a1, y1, degp, b1.reshape(1, -1), W2)
    a2 = _aggregate(y2, src3, dst3, n_acc)
    return _final_layer(a2, y2, degp, b2.reshape(1, -1))


# speedup vs baseline: 1.9288x; 1.9288x over previous
import jax
import jax.numpy as jnp
from jax import lax
from jax.experimental import pallas as pl
from jax.experimental.pallas import tpu as pltpu
from jax.experimental.pallas import tpu_sc as plsc

NC = 2
NS = 16
NW = NC * NS
LANE = 16
CHUNK = 128
TCH = 64
DEGW = 16
NB = 4
KD = 8


def _mesh():
    return plsc.VectorSubcoreMesh(
        core_axis_name="c", subcore_axis_name="s", num_cores=NC, num_subcores=NS
    )


def _deg_body(dst3, out, idx_v, ones_v, zrow_v, acc, dsem):
    c = lax.axis_index("c")
    s = lax.axis_index("s")
    w = c * NS + s
    chw = idx_v.shape[0]
    rpt = zrow_v.shape[0]

    pltpu.sync_copy(dst3.at[w], idx_v)

    @pl.loop(0, CHUNK)
    def _(r):
        ones_v[r, :] = jnp.ones((DEGW,), jnp.float32)

    @pl.loop(0, rpt)
    def _(r):
        zrow_v[r, :] = jnp.zeros((DEGW,), jnp.float32)

    pltpu.sync_copy(zrow_v, acc.at[pl.ds(s * rpt, rpt)])
    plsc.subcore_barrier()

    for j in range(KD):
        pltpu.async_copy(ones_v, acc.at[idx_v.at[j]], dsem, add=True)

    @pl.loop(0, chw - KD)
    def _(j):
        pltpu.async_copy(ones_v, acc.at[idx_v.at[j + KD]], dsem, add=True)
        pltpu.make_async_copy(ones_v, acc.at[idx_v.at[0]], dsem).wait()

    for _j in range(KD):
        pltpu.make_async_copy(ones_v, acc.at[idx_v.at[0]], dsem).wait()

    plsc.subcore_barrier()
    pltpu.sync_copy(acc.at[pl.ds(s * rpt, rpt)], out.at[c, pl.ds(s * rpt, rpt)])


def _agg_body(y, src3, dst3, out, idxs_v, idxd_v, rows_v, zbuf_v, acc,
              g0, g1, g2, g3, s0, s1, s2, s3):
    gsems = (g0, g1, g2, g3)
    ssems = (s0, s1, s2, s3)
    c = lax.axis_index("c")
    s = lax.axis_index("s")
    w = c * NS + s
    chw = idxs_v.shape[0]
    n_acc, d = acc.shape
    rpt = n_acc // NS
    zr = zbuf_v.shape[0]

    pltpu.sync_copy(src3.at[w], idxs_v)
    pltpu.sync_copy(dst3.at[w], idxd_v)

    @pl.loop(0, zr)
    def _(r):
        for q in range(d // LANE):
            zbuf_v[r, pl.ds(q * LANE, LANE)] = jnp.zeros((LANE,), jnp.float32)

    for k in range(rpt // zr):
        pltpu.async_copy(zbuf_v, acc.at[pl.ds(s * rpt + k * zr, zr)], g0)
    for k in range(rpt // zr):
        pltpu.make_async_copy(zbuf_v, acc.at[pl.ds(s * rpt, zr)], g0).wait()
    plsc.subcore_barrier()

    ng = idxs_v.shape[0] // 2

    def sidx(idx, u, q):
        return idx.at[2 * u + q // 2, pl.ds(TCH * (q % 2), TCH)]

    def start_gather(u, q):
        pltpu.async_copy(y.at[sidx(idxs_v, u, q)], rows_v.at[q], gsems[q])

    def wait_gather(q):
        pltpu.make_async_copy(
            y.at[sidx(idxs_v, 0, 0)], rows_v.at[q], gsems[q]).wait()

    def start_scatter(u, q):
        pltpu.async_copy(
            rows_v.at[q], acc.at[sidx(idxd_v, u, q)], ssems[q], add=True)

    def wait_scatter(q):
        pltpu.make_async_copy(
            rows_v.at[q], acc.at[sidx(idxd_v, 0, 0)], ssems[q]).wait()

    for p in range(2):
        pltpu.sync_copy(src3.at[2 * w + p], idxs_v)
        pltpu.sync_copy(dst3.at[2 * w + p], idxd_v)

        for q in range(NB):
            start_gather(0, q)

        @pl.loop(0, ng - 1)
        def _(u):
            for q in range(NB):
                wait_gather(q)
                start_scatter(u, q)
            for q in range(NB):
                wait_scatter(q)
                start_gather(u + 1, q)

        for q in range(NB):
            wait_gather(q)
            start_scatter(ng - 1, q)
        for q in range(NB):
            wait_scatter(q)

    plsc.subcore_barrier()
    for k in range(rpt // CHUNK):
        off = s * rpt + k * CHUNK
        pltpu.async_copy(acc.at[pl.ds(off, CHUNK)], out.at[c, pl.ds(off, CHUNK)], g0)
    for k in range(rpt // CHUNK):
        pltpu.make_async_copy(
            acc.at[pl.ds(s * rpt, CHUNK)], out.at[c, pl.ds(s * rpt, CHUNK)], g0).wait()


_SC_LINEAR = pltpu.CompilerParams(use_tc_tiling_on_sc=False)


def _count_degrees(dst3, n_acc):
    chw = dst3.shape[1]
    rpt = n_acc // NS
    return pl.kernel(
        _deg_body,
        out_type=jax.ShapeDtypeStruct((NC, n_acc, DEGW), jnp.float32),
        mesh=_mesh(),
        compiler_params=_SC_LINEAR,
        scratch_types=[
            pltpu.VMEM((chw, CHUNK), jnp.int32),
            pltpu.VMEM((CHUNK, DEGW), jnp.float32),
            pltpu.VMEM((rpt, DEGW), jnp.float32),
            pltpu.VMEM_SHARED((n_acc, DEGW), jnp.float32),
            pltpu.SemaphoreType.DMA,
        ],
    )(dst3)


def _aggregate(y, src3, dst3, n_acc):
    chw = src3.shape[1]
    d = y.shape[1]
    src3 = src3.reshape(2 * NW, chw // 2, CHUNK)
    dst3 = dst3.reshape(2 * NW, chw // 2, CHUNK)
    return pl.kernel(
        _agg_body,
        out_type=jax.ShapeDtypeStruct((NC, n_acc, d), jnp.float32),
        mesh=_mesh(),
        scratch_types=[
            pltpu.VMEM((chw // 2, CHUNK), jnp.int32),
            pltpu.VMEM((chw // 2, CHUNK), jnp.int32),
            pltpu.VMEM((NB, TCH, d), jnp.float32),
            pltpu.VMEM((32, d), jnp.float32),
            pltpu.VMEM_SHARED((n_acc, d), jnp.float32),
        ] + [pltpu.SemaphoreType.DMA] * (2 * NB),
    )(y, src3, dst3)


def _row_block(n):
    for bm in (2000, 1024, 1000, 800, 640, 512, 400, 256, 200, 128, 80, 40, 8):
        if n % bm == 0:
            return bm
    return n


def _dinv(deg_ref, bm):
    deg = deg_ref[0, :, 0:1] + deg_ref[1, :, 0:1] + 1.0
    return lax.rsqrt(deg)


def _first_layer(x, w1, degp):
    n, d_in = x.shape
    d_h = w1.shape[1]
    bm = _row_block(n)

    def body(deg_ref, x_ref, w_ref, y_ref):
        y_ref[...] = jnp.dot(
            x_ref[...], w_ref[...],
            preferred_element_type=jnp.float32,
            precision=lax.Precision.HIGHEST,
        ) * _dinv(deg_ref, bm)

    return pl.pallas_call(
        body,
        grid=(n // bm,),
        in_specs=[
            pl.BlockSpec((NC, bm, DEGW), lambda i: (0, i, 0)),
            pl.BlockSpec((bm, d_in), lambda i: (i, 0)),
            pl.BlockSpec((d_in, d_h), lambda i: (0, 0)),
        ],
        out_specs=pl.BlockSpec((bm, d_h), lambda i: (i, 0)),
        out_shape=jax.ShapeDtypeStruct((n, d_h), jnp.float32),
    )(degp, x, w1)


def _mid_layer(aggp, y1, degp, b1, w2):
    n, d = y1.shape
    bm = _row_block(n)

    def body(agg_ref, y_ref, deg_ref, b_ref, w_ref, y2_ref):
        dinv = _dinv(deg_ref, bm)
        h = dinv * (agg_ref[0] + agg_ref[1] + y_ref[...]) + b_ref[...]
        h = jnp.maximum(h, 0.0)
        y2_ref[...] = jnp.dot(
            h, w_ref[...],
            preferred_element_type=jnp.float32,
            precision=lax.Precision.HIGHEST,
        ) * dinv

    return pl.pallas_call(
        body,
        grid=(n // bm,),
        in_specs=[
            pl.BlockSpec((NC, bm, d), lambda i: (0, i, 0)),
            pl.BlockSpec((bm, d), lambda i: (i, 0)),
            pl.BlockSpec((NC, bm, DEGW), lambda i: (0, i, 0)),
            pl.BlockSpec((1, d), lambda i: (0, 0)),
            pl.BlockSpec((d, d), lambda i: (0, 0)),
        ],
        out_specs=pl.BlockSpec((bm, d), lambda i: (i, 0)),
        out_shape=jax.ShapeDtypeStruct((n, d), jnp.float32),
    )(aggp, y1, degp, b1, w2)


def _final_layer(aggp, y2, degp, b2):
    n, d = y2.shape
    bm = _row_block(n)

    def body(agg_ref, y_ref, deg_ref, b_ref, h_ref):
        h = _dinv(deg_ref, bm) * (agg_ref[0] + agg_ref[1] + y_ref[...]) \
            + b_ref[...]
        h_ref[...] = jnp.maximum(h, 0.0)

    return pl.pallas_call(
        body,
        grid=(n // bm,),
        in_specs=[
            pl.BlockSpec((NC, bm, d), lambda i: (0, i, 0)),
            pl.BlockSpec((bm, d), lambda i: (i, 0)),
            pl.BlockSpec((NC, bm, DEGW), lambda i: (0, i, 0)),
            pl.BlockSpec((1, d), lambda i: (0, 0)),
        ],
        out_specs=pl.BlockSpec((bm, d), lambda i: (i, 0)),
        out_shape=jax.ShapeDtypeStruct((n, d), jnp.float32),
    )(aggp, y2, degp, b2)


def kernel(x, edge_index, W1, b1, W2, b2):
    n = x.shape[0]
    e = edge_index.shape[1]

    grp = NW * CHUNK * 4
    e_pad = ((e + grp - 1) // grp) * grp
    chw = e_pad // (NW * CHUNK)
    n_acc = ((n + 1 + NS * CHUNK - 1) // (NS * CHUNK)) * (NS * CHUNK)

    idt = edge_index.dtype
    pad_i = jnp.arange(e_pad - e, dtype=idt)
    src3 = jnp.concatenate(
        [edge_index[0], pad_i % n]).reshape(NW, chw, CHUNK)
    dst3 = jnp.concatenate(
        [edge_index[1], n + pad_i % (n_acc - n)]).reshape(NW, chw, CHUNK)

    degp = _count_degrees(dst3, n_acc)
    y1 = _first_layer(x, W1, degp)
    a1 = _aggregate(y1, src3, dst3, n_acc)
    y2 = _mid_layer(a1, y1, degp, b1.reshape(1, -1), W2)
    a2 = _aggregate(y2, src3, dst3, n_acc)
    return _final_layer(a2, y2, degp, b2.reshape(1, -1))


# speedup vs baseline: 1.9561x; 1.0141x over previous
import jax
import jax.numpy as jnp
import numpy as np
from jax import lax
from jax.experimental import pallas as pl
from jax.experimental.pallas import tpu as pltpu
from jax.experimental.pallas import tpu_sc as plsc

NC = 2
NS = 16
NW = NC * NS
LANE = 16
CHUNK = 128
TCH = 64
DEGW = 8
NB = 4
KD = 8


def _mesh():
    return plsc.VectorSubcoreMesh(
        core_axis_name="c", subcore_axis_name="s", num_cores=NC, num_subcores=NS
    )


def _deg_body(dst3, out, idx_v, ones_v, zrow_v, acc, dsem):
    c = lax.axis_index("c")
    s = lax.axis_index("s")
    w = c * NS + s
    chw = idx_v.shape[0]
    rpt = zrow_v.shape[0]

    pltpu.sync_copy(dst3.at[w], idx_v)

    @pl.loop(0, CHUNK)
    def _(r):
        ones_v[r, :] = jnp.ones((DEGW,), jnp.float32)

    @pl.loop(0, rpt)
    def _(r):
        zrow_v[r, :] = jnp.zeros((DEGW,), jnp.float32)

    pltpu.sync_copy(zrow_v, acc.at[pl.ds(s * rpt, rpt)])
    plsc.subcore_barrier()

    for j in range(KD):
        pltpu.async_copy(ones_v, acc.at[idx_v.at[j]], dsem, add=True)

    @pl.loop(0, chw - KD)
    def _(j):
        pltpu.async_copy(ones_v, acc.at[idx_v.at[j + KD]], dsem, add=True)
        pltpu.make_async_copy(ones_v, acc.at[idx_v.at[0]], dsem).wait()

    for _j in range(KD):
        pltpu.make_async_copy(ones_v, acc.at[idx_v.at[0]], dsem).wait()

    plsc.subcore_barrier()
    pltpu.sync_copy(acc.at[pl.ds(s * rpt, rpt)], out.at[c, pl.ds(s * rpt, rpt)])


def _agg_body(y, src3, dst3, out, idxs_v, idxd_v, rows_v, zbuf_v, acc,
              g0, g1, g2, g3, s0, s1, s2, s3):
    gsems = (g0, g1, g2, g3)
    ssems = (s0, s1, s2, s3)
    c = lax.axis_index("c")
    s = lax.axis_index("s")
    w = c * NS + s
    chw = idxs_v.shape[0]
    n_acc, d = acc.shape
    rpt = n_acc // NS
    zr = zbuf_v.shape[0]

    pltpu.sync_copy(src3.at[w], idxs_v)
    pltpu.sync_copy(dst3.at[w], idxd_v)

    @pl.loop(0, zr)
    def _(r):
        for q in range(d // LANE):
            zbuf_v[r, pl.ds(q * LANE, LANE)] = jnp.zeros((LANE,), jnp.float32)

    for k in range(rpt // zr):
        pltpu.async_copy(zbuf_v, acc.at[pl.ds(s * rpt + k * zr, zr)], g0)
    for k in range(rpt // zr):
        pltpu.make_async_copy(zbuf_v, acc.at[pl.ds(s * rpt, zr)], g0).wait()
    plsc.subcore_barrier()

    ng = idxs_v.shape[0] // 2

    def sidx(idx, u, q):
        return idx.at[2 * u + q // 2, pl.ds(TCH * (q % 2), TCH)]

    def start_gather(u, q):
        pltpu.async_copy(y.at[sidx(idxs_v, u, q)], rows_v.at[q], gsems[q])

    def wait_gather(q):
        pltpu.make_async_copy(
            y.at[sidx(idxs_v, 0, 0)], rows_v.at[q], gsems[q]).wait()

    def start_scatter(u, q):
        pltpu.async_copy(
            rows_v.at[q], acc.at[sidx(idxd_v, u, q)], ssems[q], add=True)

    def wait_scatter(q):
        pltpu.make_async_copy(
            rows_v.at[q], acc.at[sidx(idxd_v, 0, 0)], ssems[q]).wait()

    for p in range(2):
        pltpu.sync_copy(src3.at[2 * w + p], idxs_v)
        pltpu.sync_copy(dst3.at[2 * w + p], idxd_v)

        for q in range(NB):
            start_gather(0, q)

        @pl.loop(0, ng - 1)
        def _(u):
            for q in range(NB):
                wait_gather(q)
                start_scatter(u, q)
            for q in range(NB):
                wait_scatter(q)
                start_gather(u + 1, q)

        for q in range(NB):
            wait_gather(q)
            start_scatter(ng - 1, q)
        for q in range(NB):
            wait_scatter(q)

    plsc.subcore_barrier()
    for k in range(rpt // CHUNK):
        off = s * rpt + k * CHUNK
        pltpu.async_copy(acc.at[pl.ds(off, CHUNK)], out.at[c, pl.ds(off, CHUNK)], g0)
    for k in range(rpt // CHUNK):
        pltpu.make_async_copy(
            acc.at[pl.ds(s * rpt, CHUNK)], out.at[c, pl.ds(s * rpt, CHUNK)], g0).wait()


_SC_LINEAR = pltpu.CompilerParams(use_tc_tiling_on_sc=False)


def _count_degrees(dst3, n_acc):
    chw = dst3.shape[1]
    rpt = n_acc // NS
    return pl.kernel(
        _deg_body,
        out_type=jax.ShapeDtypeStruct((NC, n_acc, DEGW), jnp.float32),
        mesh=_mesh(),
        compiler_params=_SC_LINEAR,
        scratch_types=[
            pltpu.VMEM((chw, CHUNK), jnp.int32),
            pltpu.VMEM((CHUNK, DEGW), jnp.float32),
            pltpu.VMEM((rpt, DEGW), jnp.float32),
            pltpu.VMEM_SHARED((n_acc, DEGW), jnp.float32),
            pltpu.SemaphoreType.DMA,
        ],
    )(dst3)


def _aggregate(y, src3, dst3, n_acc):
    chw = src3.shape[1]
    d = y.shape[1]
    src3 = src3.reshape(2 * NW, chw // 2, CHUNK)
    dst3 = dst3.reshape(2 * NW, chw // 2, CHUNK)
    return pl.kernel(
        _agg_body,
        out_type=jax.ShapeDtypeStruct((NC, n_acc, d), jnp.float32),
        mesh=_mesh(),
        scratch_types=[
            pltpu.VMEM((chw // 2, CHUNK), jnp.int32),
            pltpu.VMEM((chw // 2, CHUNK), jnp.int32),
            pltpu.VMEM((NB, TCH, d), jnp.float32),
            pltpu.VMEM((32, d), jnp.float32),
            pltpu.VMEM_SHARED((n_acc, d), jnp.float32),
        ] + [pltpu.SemaphoreType.DMA] * (2 * NB),
    )(y, src3, dst3)


def _row_block(n):
    for bm in (2000, 1024, 1000, 800, 640, 512, 400, 256, 200, 128, 80, 40, 8):
        if n % bm == 0:
            return bm
    return n


def _dinv(deg_ref, bm):
    deg = deg_ref[0, :, 0:1] + deg_ref[1, :, 0:1] + 1.0
    return lax.rsqrt(deg)


def _matmul(x, w1):
    n, d_in = x.shape
    d_h = w1.shape[1]
    bm = _row_block(n)

    def body(x_ref, w_ref, y_ref):
        y_ref[...] = jnp.dot(
            x_ref[...], w_ref[...],
            preferred_element_type=jnp.float32,
            precision=lax.Precision.HIGHEST,
        )

    return pl.pallas_call(
        body,
        grid=(n // bm,),
        in_specs=[
            pl.BlockSpec((bm, d_in), lambda i: (i, 0)),
            pl.BlockSpec((d_in, d_h), lambda i: (0, 0)),
        ],
        out_specs=pl.BlockSpec((bm, d_h), lambda i: (i, 0)),
        out_shape=jax.ShapeDtypeStruct((n, d_h), jnp.float32),
    )(x, w1)


def _scale(xw, degp):
    n, d_h = xw.shape
    bm = _row_block(n)

    def body(deg_ref, xw_ref, y_ref):
        y_ref[...] = xw_ref[...] * _dinv(deg_ref, bm)

    return pl.pallas_call(
        body,
        grid=(n // bm,),
        in_specs=[
            pl.BlockSpec((NC, bm, DEGW), lambda i: (0, i, 0)),
            pl.BlockSpec((bm, d_h), lambda i: (i, 0)),
        ],
        out_specs=pl.BlockSpec((bm, d_h), lambda i: (i, 0)),
        out_shape=jax.ShapeDtypeStruct((n, d_h), jnp.float32),
    )(degp, xw)


def _mid_layer(aggp, y1, degp, b1, w2):
    n, d = y1.shape
    bm = _row_block(n)

    def body(agg_ref, y_ref, deg_ref, b_ref, w_ref, y2_ref):
        dinv = _dinv(deg_ref, bm)
        h = dinv * (agg_ref[0] + agg_ref[1] + y_ref[...]) + b_ref[...]
        h = jnp.maximum(h, 0.0)
        y2_ref[...] = jnp.dot(
            h, w_ref[...],
            preferred_element_type=jnp.float32,
            precision=lax.Precision.HIGHEST,
        ) * dinv

    return pl.pallas_call(
        body,
        grid=(n // bm,),
        in_specs=[
            pl.BlockSpec((NC, bm, d), lambda i: (0, i, 0)),
            pl.BlockSpec((bm, d), lambda i: (i, 0)),
            pl.BlockSpec((NC, bm, DEGW), lambda i: (0, i, 0)),
            pl.BlockSpec((1, d), lambda i: (0, 0)),
            pl.BlockSpec((d, d), lambda i: (0, 0)),
        ],
        out_specs=pl.BlockSpec((bm, d), lambda i: (i, 0)),
        out_shape=jax.ShapeDtypeStruct((n, d), jnp.float32),
    )(aggp, y1, degp, b1, w2)


def _final_layer(aggp, y2, degp, b2):
    n, d = y2.shape
    bm = _row_block(n)

    def body(agg_ref, y_ref, deg_ref, b_ref, h_ref):
        h = _dinv(deg_ref, bm) * (agg_ref[0] + agg_ref[1] + y_ref[...]) \
            + b_ref[...]
        h_ref[...] = jnp.maximum(h, 0.0)

    return pl.pallas_call(
        body,
        grid=(n // bm,),
        in_specs=[
            pl.BlockSpec((NC, bm, d), lambda i: (0, i, 0)),
            pl.BlockSpec((bm, d), lambda i: (i, 0)),
            pl.BlockSpec((NC, bm, DEGW), lambda i: (0, i, 0)),
            pl.BlockSpec((1, d), lambda i: (0, 0)),
        ],
        out_specs=pl.BlockSpec((bm, d), lambda i: (i, 0)),
        out_shape=jax.ShapeDtypeStruct((n, d), jnp.float32),
    )(aggp, y2, degp, b2)


def kernel(x, edge_index, W1, b1, W2, b2):
    n = x.shape[0]
    e = edge_index.shape[1]

    grp = NW * CHUNK * 4
    e_pad = ((e + grp - 1) // grp) * grp
    chw = e_pad // (NW * CHUNK)
    n_acc = ((n + 1 + NS * CHUNK - 1) // (NS * CHUNK)) * (NS * CHUNK)

    pad_i = np.arange(e_pad - e, dtype=np.int32)
    src3 = jnp.concatenate(
        [edge_index[0], jnp.asarray(pad_i % n)]).reshape(NW, chw, CHUNK)
    dst3 = jnp.concatenate(
        [edge_index[1], jnp.asarray(n + pad_i % (n_acc - n))]).reshape(NW, chw, CHUNK)

    degp = _count_degrees(dst3, n_acc)
    y1 = _scale(_matmul(x, W1), degp)
    a1 = _aggregate(y1, src3, dst3, n_acc)
    y2 = _mid_layer(a1, y1, degp, b1.reshape(1, -1), W2)
    a2 = _aggregate(y2, src3, dst3, n_acc)
    return _final_layer(a2, y2, degp, b2.reshape(1, -1))
